# baseline (speedup 1.0000x reference)
import functools

import jax
import jax.numpy as jnp
from jax.experimental import pallas as pl
from jax.experimental.pallas import tpu as pltpu


_SQRT_2_OVER_PI = 0.7978845608028654


def _gate_from(slab, w1t_ref, w2_ref, inv_n):
    pooled = jnp.sum(slab, axis=-1, keepdims=True) * inv_n
    h = jnp.sum(w1t_ref[...] * pooled, axis=0, keepdims=True)
    h = 0.5 * h * (1.0 + jnp.tanh(_SQRT_2_OVER_PI * (h + 0.044715 * (h * h * h))))
    g = jnp.sum(w2_ref[...] * h, axis=1, keepdims=True)
    return 0.5 * (1.0 + jnp.tanh(0.5 * g))


def _se3d_body(xa_ref, xb_ref, w1t_ref, w2_ref, o_ref, *, inv_n):
    s = pl.program_id(1)

    @pl.when(s == 0)
    def _():
        o_ref[0, 0] = xa_ref[0, 0] * _gate_from(xa_ref[0, 0], w1t_ref,
                                                w2_ref, inv_n)

    @pl.when(s == 1)
    def _():
        o_ref[0, 0] = xb_ref[0, 0] * _gate_from(xb_ref[0, 0], w1t_ref,
                                                w2_ref, inv_n)


def kernel(x, w1, w2):
    B, C, D, H, W = x.shape
    N = D * H * W
    hidden = w1.shape[0]
    hb = B // 2

    x4 = x.reshape(2, hb, C, N)
    w1t = jnp.transpose(w1)

    out4 = pl.pallas_call(
        functools.partial(_se3d_body, inv_n=1.0 / N),
        out_shape=jax.ShapeDtypeStruct((2, hb, C, N), x.dtype),
        grid=(hb, 2),
        in_specs=[
            pl.BlockSpec((1, 1, C, N), lambda b, s: (0, b, 0, 0)),
            pl.BlockSpec((1, 1, C, N), lambda b, s: (1, b, 0, 0)),
            pl.BlockSpec((C, hidden), lambda b, s: (0, 0)),
            pl.BlockSpec((C, hidden), lambda b, s: (0, 0)),
        ],
        out_specs=pl.BlockSpec((1, 1, C, N), lambda b, s: (s, b, 0, 0)),
        compiler_params=pltpu.CompilerParams(
            dimension_semantics=("parallel", "arbitrary"),
            vmem_limit_bytes=48 << 20,
        ),
    )(x4, x4, w1t, w2)
    return out4.reshape(B, C, D, H, W)


# speedup vs baseline: 1.0784x; 1.0784x over previous
import functools

import jax
import jax.numpy as jnp
from jax.experimental import pallas as pl
from jax.experimental.pallas import tpu as pltpu


_SQRT_2_OVER_PI = 0.7978845608028654


def _se3d_body(xt_ref, xb_ref, w1t_ref, w2_ref, o_ref, g_ref, *, inv_n, hc):
    xt = xt_ref[0]
    xb = xb_ref[0]
    pt = jnp.sum(xt, axis=-1, keepdims=True) * inv_n
    pb = jnp.sum(xb, axis=-1, keepdims=True) * inv_n
    h = (jnp.sum(w1t_ref[0:hc] * pt, axis=0, keepdims=True)
         + jnp.sum(w1t_ref[hc:] * pb, axis=0, keepdims=True))
    h = 0.5 * h * (1.0 + jnp.tanh(_SQRT_2_OVER_PI * (h + 0.044715 * (h * h * h))))
    g = jnp.sum(w2_ref[...] * h, axis=1, keepdims=True)
    gate = 0.5 * (1.0 + jnp.tanh(0.5 * g))
    o_ref[0, :hc] = xt * gate[0:hc]
    o_ref[0, hc:] = xb * gate[hc:]
    g_ref[0] = gate


def kernel(x, w1, w2):
    B, C, D, H, W = x.shape
    N = D * H * W
    hidden = w1.shape[0]
    hc = C // 2

    x3 = x.reshape(B, C, N)
    w1t = jnp.transpose(w1)

    out3, _gates = pl.pallas_call(
        functools.partial(_se3d_body, inv_n=1.0 / N, hc=hc),
        out_shape=[jax.ShapeDtypeStruct((B, C, N), x.dtype),
                   jax.ShapeDtypeStruct((B, C, 1), jnp.float32)],
        grid=(B,),
        in_specs=[
            pl.BlockSpec((1, hc, N), lambda b: (b, 0, 0)),
            pl.BlockSpec((1, hc, N), lambda b: (b, 1, 0)),
            pl.BlockSpec((C, hidden), lambda b: (0, 0)),
            pl.BlockSpec((C, hidden), lambda b: (0, 0)),
        ],
        out_specs=[pl.BlockSpec((1, C, N), lambda b: (b, 0, 0)),
                   pl.BlockSpec((1, C, 1), lambda b: (b, 0, 0))],
        compiler_params=pltpu.CompilerParams(
            dimension_semantics=("parallel",),
            vmem_limit_bytes=48 << 20,
        ),
    )(x3, x3, w1t, w2)
    return out3.reshape(B, C, D, H, W)
